# Initial kernel scaffold; baseline (speedup 1.0000x reference)
import jax
import jax.numpy as jnp
from jax.experimental import pallas as pl


def kernel(pattern, pattern_gt, mask):
    raise NotImplementedError("write your pallas kernel here")


# speedup vs baseline: 27.5796x; 27.5796x over previous
import functools

import jax
import jax.numpy as jnp
from jax.experimental import pallas as pl

_H = 384
_W = 384
_N = float(_H * _W)


def _body(x_ref, gt_ref, m_ref, out_ref):
    i = pl.program_id(0)
    v = x_ref[0]
    m = m_ref[...]
    g = gt_ref[...]
    m2 = m * m
    m2g = m2 * g

    S = jnp.sum(v)
    A = jnp.sum(v * v * m2)
    B = jnp.sum(v * m2)
    D = jnp.sum(v * m2g)
    C = jnp.sum(m2)
    E = jnp.sum(m2g)
    F = jnp.sum(m2g * g)

    def step(_, carry):
        vv, acc, rem = carry
        mx = jnp.max(vv)
        cnt = jnp.sum(jnp.where(vv == mx, 1.0, 0.0))
        take = jnp.minimum(cnt, rem)
        acc = acc + jnp.where(take > 0.0, take * mx, 0.0)
        rem = rem - take
        vv = jnp.where(vv == mx, -jnp.inf, vv)
        return vv, acc, rem

    _, top10_sum, _ = jax.lax.fori_loop(
        0, 10, step, (v, jnp.float32(0.0), jnp.float32(10.0))
    )

    mu = S / _N
    max_avg = top10_sum / 10.0 - mu
    denom = jnp.where(max_avg < 1e-20, max_avg + 1e-19, max_avg)
    num = (A - 2.0 * mu * B + mu * mu * C) / (denom * denom) \
        - 2.0 * (D - mu * E) / denom + F
    loss_i = num / _N

    @pl.when(i == 0)
    def _():
        out_ref[...] = jnp.zeros_like(out_ref)

    out_ref[...] += jnp.full(out_ref.shape, loss_i, dtype=jnp.float32)


@jax.jit
def kernel(pattern, pattern_gt, mask):
    ch = pattern.shape[1]
    x = pattern.reshape(ch, _H, _W)
    out = pl.pallas_call(
        _body,
        grid=(ch,),
        in_specs=[
            pl.BlockSpec((1, _H, _W), lambda i: (i, 0, 0)),
            pl.BlockSpec((_H, _W), lambda i: (0, 0)),
            pl.BlockSpec((_H, _W), lambda i: (0, 0)),
        ],
        out_specs=pl.BlockSpec((8, 128), lambda i: (0, 0)),
        out_shape=jax.ShapeDtypeStruct((8, 128), jnp.float32),
    )(x, pattern_gt, mask)
    return out[0, 0].reshape(1)


# speedup vs baseline: 60.3731x; 2.1891x over previous
import jax
import jax.numpy as jnp
from jax.experimental import pallas as pl
from jax.experimental.pallas import tpu as pltpu

_H = 384
_W = 384
_N = float(_H * _W)


def _body(x_ref, gt_ref, m_ref, out_ref, m2_ref, m2g_ref, cef_ref):
    i = pl.program_id(0)

    @pl.when(i == 0)
    def _():
        m = m_ref[...]
        g = gt_ref[...]
        m2 = m * m
        m2g = m2 * g
        m2_ref[...] = m2
        m2g_ref[...] = m2g
        cef_ref[0] = jnp.sum(m2)
        cef_ref[1] = jnp.sum(m2g)
        cef_ref[2] = jnp.sum(m2g * g)
        out_ref[...] = jnp.zeros(out_ref.shape, jnp.float32)

    v = x_ref[0]
    m2 = m2_ref[...]
    m2g = m2g_ref[...]

    S = jnp.sum(v)
    A = jnp.sum(v * v * m2)
    B = jnp.sum(v * m2)
    D = jnp.sum(v * m2g)

    M = jnp.max(v.reshape(_H // 8, 8, _W), axis=0)

    def tau_step(_, carry):
        MM, tau, rem = carry
        mx = jnp.max(MM)
        cnt = jnp.sum(jnp.where(MM == mx, 1.0, 0.0))
        tau = jnp.where(rem > 0.0, mx, tau)
        rem = rem - jnp.minimum(cnt, rem)
        MM = jnp.where(MM == mx, -jnp.inf, MM)
        return MM, tau, rem

    _, tau, _ = jax.lax.fori_loop(
        0, 10, tau_step, (M, jnp.float32(-jnp.inf), jnp.float32(10.0))
    )

    sel = v >= tau
    cnt = jnp.sum(jnp.where(sel, 1.0, 0.0))
    ssum = jnp.sum(jnp.where(sel, v, 0.0))

    def exact_fallback(_):
        def step(_, carry):
            bound, acc, rem = carry
            w = jnp.where((v >= tau) & (v < bound), v, -jnp.inf)
            mx = jnp.max(w)
            c = jnp.sum(jnp.where(w == mx, 1.0, 0.0))
            take = jnp.minimum(c, rem)
            acc = acc + jnp.where(take > 0.0, take * mx, 0.0)
            rem = rem - take
            return mx, acc, rem

        _, acc, _ = jax.lax.fori_loop(
            0, 10, step, (jnp.float32(jnp.inf), jnp.float32(0.0),
                          jnp.float32(10.0))
        )
        return acc

    top10_sum = jax.lax.cond(cnt == 10.0, lambda _: ssum, exact_fallback,
                             operand=None)

    C = cef_ref[0]
    E = cef_ref[1]
    F = cef_ref[2]

    mu = S / _N
    max_avg = top10_sum / 10.0 - mu
    denom = jnp.where(max_avg < 1e-20, max_avg + 1e-19, max_avg)
    num = ((A - 2.0 * mu * B + mu * mu * C) / denom
           - 2.0 * (D - mu * E)) / denom + F
    loss_i = num / _N

    out_ref[...] += jnp.full(out_ref.shape, loss_i, dtype=jnp.float32)


@jax.jit
def kernel(pattern, pattern_gt, mask):
    ch = pattern.shape[1]
    x = pattern.reshape(ch, _H, _W)
    out = pl.pallas_call(
        _body,
        grid=(ch,),
        in_specs=[
            pl.BlockSpec((1, _H, _W), lambda i: (i, 0, 0)),
            pl.BlockSpec((_H, _W), lambda i: (0, 0)),
            pl.BlockSpec((_H, _W), lambda i: (0, 0)),
        ],
        out_specs=pl.BlockSpec((8, 128), lambda i: (0, 0)),
        out_shape=jax.ShapeDtypeStruct((8, 128), jnp.float32),
        scratch_shapes=[
            pltpu.VMEM((_H, _W), jnp.float32),
            pltpu.VMEM((_H, _W), jnp.float32),
            pltpu.SMEM((3,), jnp.float32),
        ],
    )(x, pattern_gt, mask)
    return out[0, 0].reshape(1)


# speedup vs baseline: 79.5759x; 1.3181x over previous
import jax
import jax.numpy as jnp
from jax.experimental import pallas as pl
from jax.experimental.pallas import tpu as pltpu

_H = 384
_W = 384
_N = float(_H * _W)
_R = _H // 8


def _body(x_ref, gt_ref, m_ref, out_ref, m2_ref, m2g_ref, cef_ref):
    i = pl.program_id(0)

    @pl.when(i == 0)
    def _():
        m = m_ref[0]
        g = gt_ref[0]
        m2 = m * m
        m2g = m2 * g
        m2_ref[0] = m2
        m2g_ref[0] = m2g
        cef_ref[0] = jnp.sum(m2)
        cef_ref[1] = jnp.sum(m2g)
        cef_ref[2] = jnp.sum(m2g * g)
        out_ref[...] = jnp.zeros(out_ref.shape, jnp.float32)

    def chunk(j, carry):
        aS, aA, aB, aD, aM = carry
        xv = x_ref[0, j]
        m2c = m2_ref[0, j]
        m2gc = m2g_ref[0, j]
        vm2 = xv * m2c
        aS = aS + xv
        aA = aA + xv * vm2
        aB = aB + vm2
        aD = aD + xv * m2gc
        aM = jnp.maximum(aM, xv)
        return aS, aA, aB, aD, aM

    zero = jnp.zeros((8, _W), jnp.float32)
    aS, aA, aB, aD, M = jax.lax.fori_loop(
        0, _R, chunk,
        (zero, zero, zero, zero, jnp.full((8, _W), -jnp.inf, jnp.float32)),
        unroll=4,
    )
    S = jnp.sum(aS)
    A = jnp.sum(aA)
    B = jnp.sum(aB)
    D = jnp.sum(aD)

    def tau_step(_, carry):
        MM, tau = carry
        mx = jnp.max(MM)
        MM = jnp.where(MM == mx, -jnp.inf, MM)
        return MM, mx

    _, tau = jax.lax.fori_loop(
        0, 10, tau_step, (M, jnp.float32(-jnp.inf))
    )

    def fchunk(j, carry):
        aC, aV = carry
        xv = x_ref[0, j]
        sel = xv >= tau
        aC = aC + jnp.where(sel, 1.0, 0.0)
        aV = aV + jnp.where(sel, xv, 0.0)
        return aC, aV

    aC, aV = jax.lax.fori_loop(0, _R, fchunk, (zero, zero), unroll=4)
    cnt = jnp.sum(aC)
    ssum = jnp.sum(aV)

    def exact_fallback(_):
        def step(_, carry):
            bound, acc, rem = carry
            v = x_ref[0]
            w = jnp.where((v >= tau) & (v < bound), v, -jnp.inf)
            mx = jnp.max(w)
            c = jnp.sum(jnp.where(w == mx, 1.0, 0.0))
            take = jnp.minimum(c, rem)
            acc = acc + jnp.where(take > 0.0, take * mx, 0.0)
            rem = rem - take
            return mx, acc, rem

        _, acc, _ = jax.lax.fori_loop(
            0, 10, step, (jnp.float32(jnp.inf), jnp.float32(0.0),
                          jnp.float32(10.0))
        )
        return acc

    top10_sum = jax.lax.cond(cnt == 10.0, lambda _: ssum, exact_fallback,
                             operand=None)

    C = cef_ref[0]
    E = cef_ref[1]
    F = cef_ref[2]

    mu = S / _N
    max_avg = top10_sum / 10.0 - mu
    denom = jnp.where(max_avg < 1e-20, max_avg + 1e-19, max_avg)
    num = ((A - 2.0 * mu * B + mu * mu * C) / denom
           - 2.0 * (D - mu * E)) / denom + F
    loss_i = num / _N

    out_ref[...] += jnp.full(out_ref.shape, loss_i, dtype=jnp.float32)


@jax.jit
def kernel(pattern, pattern_gt, mask):
    ch = pattern.shape[1]
    x = pattern.reshape(ch, _R, 8, _W)
    out = pl.pallas_call(
        _body,
        grid=(ch,),
        in_specs=[
            pl.BlockSpec((1, _R, 8, _W), lambda i: (i, 0, 0, 0)),
            pl.BlockSpec((1, _R, 8, _W), lambda i: (0, 0, 0, 0)),
            pl.BlockSpec((1, _R, 8, _W), lambda i: (0, 0, 0, 0)),
        ],
        out_specs=pl.BlockSpec((8, 128), lambda i: (0, 0)),
        out_shape=jax.ShapeDtypeStruct((8, 128), jnp.float32),
        scratch_shapes=[
            pltpu.VMEM((1, _R, 8, _W), jnp.float32),
            pltpu.VMEM((1, _R, 8, _W), jnp.float32),
            pltpu.SMEM((3,), jnp.float32),
        ],
    )(x, pattern_gt.reshape(1, _R, 8, _W), mask.reshape(1, _R, 8, _W))
    return out[0, 0].reshape(1)


# speedup vs baseline: 83.4724x; 1.0490x over previous
import jax
import jax.numpy as jnp
from jax.experimental import pallas as pl
from jax.experimental.pallas import tpu as pltpu

_H = 384
_W = 384
_N = float(_H * _W)
_R = _H // 8
_CPB = 2


def _pair_scalars(x_ref, m2_ref, m2g_ref):
    def chunk(j, carry):
        accs = []
        m2c = m2_ref[0, j]
        m2gc = m2g_ref[0, j]
        for c in range(_CPB):
            aS, aA, aB, aD, aM = carry[c]
            xv = x_ref[c, j]
            vm2 = xv * m2c
            aS = aS + xv
            aA = aA + xv * vm2
            aB = aB + vm2
            aD = aD + xv * m2gc
            aM = jnp.maximum(aM, xv)
            accs.append((aS, aA, aB, aD, aM))
        return tuple(accs)

    zero = jnp.zeros((8, _W), jnp.float32)
    init = tuple(
        (zero, zero, zero, zero,
         jnp.full((8, _W), -jnp.inf, jnp.float32))
        for _ in range(_CPB)
    )
    final = jax.lax.fori_loop(0, _R, chunk, init, unroll=True)
    return [
        (jnp.sum(aS), jnp.sum(aA), jnp.sum(aB), jnp.sum(aD), M)
        for aS, aA, aB, aD, M in final
    ]


def _top10_sum(x_ref, c, M):
    def tau_step(_, carry):
        MM, tau = carry
        mx = jnp.max(MM)
        MM = jnp.where(MM == mx, -jnp.inf, MM)
        return MM, mx

    _, tau = jax.lax.fori_loop(0, 10, tau_step, (M, jnp.float32(-jnp.inf)))

    def fchunk(j, carry):
        aC, aV = carry
        xv = x_ref[c, j]
        sel = xv >= tau
        aC = aC + jnp.where(sel, 1.0, 0.0)
        aV = aV + jnp.where(sel, xv, 0.0)
        return aC, aV

    zero = jnp.zeros((8, _W), jnp.float32)
    aC, aV = jax.lax.fori_loop(0, _R, fchunk, (zero, zero), unroll=True)
    cnt = jnp.sum(aC)
    ssum = jnp.sum(aV)

    def exact_fallback(_):
        def step(_, carry):
            bound, acc, rem = carry
            v = x_ref[c]
            w = jnp.where((v >= tau) & (v < bound), v, -jnp.inf)
            mx = jnp.max(w)
            cc = jnp.sum(jnp.where(w == mx, 1.0, 0.0))
            take = jnp.minimum(cc, rem)
            acc = acc + jnp.where(take > 0.0, take * mx, 0.0)
            rem = rem - take
            return mx, acc, rem

        _, acc, _ = jax.lax.fori_loop(
            0, 10, step, (jnp.float32(jnp.inf), jnp.float32(0.0),
                          jnp.float32(10.0))
        )
        return acc

    return jax.lax.cond(cnt == 10.0, lambda _: ssum, exact_fallback,
                        operand=None)


def _body(x_ref, gt_ref, m_ref, out_ref, m2_ref, m2g_ref, cef_ref):
    i = pl.program_id(0)

    @pl.when(i == 0)
    def _():
        m = m_ref[0]
        g = gt_ref[0]
        m2 = m * m
        m2g = m2 * g
        m2_ref[0] = m2
        m2g_ref[0] = m2g
        cef_ref[0] = jnp.sum(m2)
        cef_ref[1] = jnp.sum(m2g)
        cef_ref[2] = jnp.sum(m2g * g)
        out_ref[...] = jnp.zeros(out_ref.shape, jnp.float32)

    C = cef_ref[0]
    E = cef_ref[1]
    F = cef_ref[2]

    loss = jnp.float32(0.0)
    per_channel = _pair_scalars(x_ref, m2_ref, m2g_ref)
    for c in range(_CPB):
        S, A, B, D, M = per_channel[c]
        top10_sum = _top10_sum(x_ref, c, M)
        mu = S / _N
        max_avg = top10_sum / 10.0 - mu
        denom = jnp.where(max_avg < 1e-20, max_avg + 1e-19, max_avg)
        num = ((A - 2.0 * mu * B + mu * mu * C) / denom
               - 2.0 * (D - mu * E)) / denom + F
        loss = loss + num / _N

    out_ref[...] += jnp.full(out_ref.shape, loss, dtype=jnp.float32)


@jax.jit
def kernel(pattern, pattern_gt, mask):
    ch = pattern.shape[1]
    x = pattern.reshape(ch, _R, 8, _W)
    out = pl.pallas_call(
        _body,
        grid=(ch // _CPB,),
        in_specs=[
            pl.BlockSpec((_CPB, _R, 8, _W), lambda i: (i, 0, 0, 0)),
            pl.BlockSpec((1, _R, 8, _W), lambda i: (0, 0, 0, 0)),
            pl.BlockSpec((1, _R, 8, _W), lambda i: (0, 0, 0, 0)),
        ],
        out_specs=pl.BlockSpec((8, 128), lambda i: (0, 0)),
        out_shape=jax.ShapeDtypeStruct((8, 128), jnp.float32),
        scratch_shapes=[
            pltpu.VMEM((1, _R, 8, _W), jnp.float32),
            pltpu.VMEM((1, _R, 8, _W), jnp.float32),
            pltpu.SMEM((3,), jnp.float32),
        ],
    )(x, pattern_gt.reshape(1, _R, 8, _W), mask.reshape(1, _R, 8, _W))
    return out[0, 0].reshape(1)


# speedup vs baseline: 146.6699x; 1.7571x over previous
import jax
import jax.numpy as jnp
from jax.experimental import pallas as pl
from jax.experimental.pallas import tpu as pltpu

_H = 384
_W = 384
_N = float(_H * _W)
_R = _H // 8
_CPB = 4


def _top10_sum_fallback(x_ref, c, tau):
    def step(_, carry):
        bound, acc, rem = carry
        v = x_ref[c]
        w = jnp.where((v >= tau) & (v < bound), v, -jnp.inf)
        mx = jnp.max(w)
        cc = jnp.sum(jnp.where(w == mx, 1.0, 0.0))
        take = jnp.minimum(cc, rem)
        acc = acc + jnp.where(take > 0.0, take * mx, 0.0)
        rem = rem - take
        return mx, acc, rem

    _, acc, _ = jax.lax.fori_loop(
        0, 10, step,
        (jnp.float32(jnp.inf), jnp.float32(0.0), jnp.float32(10.0))
    )
    return acc


def _body(x_ref, gt_ref, m_ref, out_ref, m2_ref, m2g_ref, cef_ref):
    i = pl.program_id(0)

    @pl.when(i == 0)
    def _():
        m = m_ref[0]
        g = gt_ref[0]
        m2 = m * m
        m2g = m2 * g
        m2_ref[0] = m2
        m2g_ref[0] = m2g
        cef_ref[0] = jnp.sum(m2)
        cef_ref[1] = jnp.sum(m2g)
        cef_ref[2] = jnp.sum(m2g * g)
        out_ref[...] = jnp.zeros(out_ref.shape, jnp.float32)

    def chunk(j, carry):
        m2c = m2_ref[0, j]
        m2gc = m2g_ref[0, j]
        accs = []
        for c in range(_CPB):
            aS, aA, aB, aD, aM = carry[c]
            xv = x_ref[c, j]
            vm2 = xv * m2c
            aS = aS + xv
            aA = aA + xv * vm2
            aB = aB + vm2
            aD = aD + xv * m2gc
            aM = jnp.maximum(aM, xv)
            accs.append((aS, aA, aB, aD, aM))
        return tuple(accs)

    zero = jnp.zeros((8, _W), jnp.float32)
    init = tuple(
        (zero, zero, zero, zero,
         jnp.full((8, _W), -jnp.inf, jnp.float32))
        for _ in range(_CPB)
    )
    final = jax.lax.fori_loop(0, _R, chunk, init, unroll=True)

    sums = []
    Ws = []
    for c in range(_CPB):
        aS, aA, aB, aD, M = final[c]
        sums.append((jnp.sum(aS), jnp.sum(aA), jnp.sum(aB), jnp.sum(aD)))
        Ws.append(jnp.maximum(jnp.maximum(M[:, :128], M[:, 128:256]),
                              M[:, 256:]))

    def tau_step(_, carry):
        out = []
        for c in range(_CPB):
            W, tau = carry[c]
            mx = jnp.max(W)
            W = jnp.where(W == mx, -jnp.inf, W)
            out.append((W, mx))
        return tuple(out)

    taus_c = jax.lax.fori_loop(
        0, 10, tau_step,
        tuple((Ws[c], jnp.float32(-jnp.inf)) for c in range(_CPB)),
        unroll=True,
    )
    taus = [taus_c[c][1] for c in range(_CPB)]

    def fchunk(j, carry):
        out = []
        for c in range(_CPB):
            aC, aV = carry[c]
            xv = x_ref[c, j]
            sel = xv >= taus[c]
            aC = aC + jnp.where(sel, 1.0, 0.0)
            aV = aV + jnp.where(sel, xv, 0.0)
            out.append((aC, aV))
        return tuple(out)

    facc = jax.lax.fori_loop(
        0, _R, fchunk, tuple((zero, zero) for _ in range(_CPB)),
        unroll=True,
    )

    C = cef_ref[0]
    E = cef_ref[1]
    F = cef_ref[2]

    loss = jnp.float32(0.0)
    for c in range(_CPB):
        cnt = jnp.sum(facc[c][0])
        ssum = jnp.sum(facc[c][1])
        tau = taus[c]
        top10_sum = jax.lax.cond(
            cnt == 10.0, lambda _: ssum,
            lambda _: _top10_sum_fallback(x_ref, c, tau), operand=None)
        S, A, B, D = sums[c]
        mu = S / _N
        max_avg = top10_sum / 10.0 - mu
        denom = jnp.where(max_avg < 1e-20, max_avg + 1e-19, max_avg)
        num = ((A - 2.0 * mu * B + mu * mu * C) / denom
               - 2.0 * (D - mu * E)) / denom + F
        loss = loss + num / _N

    out_ref[...] += jnp.full(out_ref.shape, loss, dtype=jnp.float32)


@jax.jit
def kernel(pattern, pattern_gt, mask):
    ch = pattern.shape[1]
    x = pattern.reshape(ch, _R, 8, _W)
    out = pl.pallas_call(
        _body,
        grid=(ch // _CPB,),
        in_specs=[
            pl.BlockSpec((_CPB, _R, 8, _W), lambda i: (i, 0, 0, 0)),
            pl.BlockSpec((1, _R, 8, _W), lambda i: (0, 0, 0, 0)),
            pl.BlockSpec((1, _R, 8, _W), lambda i: (0, 0, 0, 0)),
        ],
        out_specs=pl.BlockSpec((8, 128), lambda i: (0, 0)),
        out_shape=jax.ShapeDtypeStruct((8, 128), jnp.float32),
        scratch_shapes=[
            pltpu.VMEM((1, _R, 8, _W), jnp.float32),
            pltpu.VMEM((1, _R, 8, _W), jnp.float32),
            pltpu.SMEM((3,), jnp.float32),
        ],
    )(x, pattern_gt.reshape(1, _R, 8, _W), mask.reshape(1, _R, 8, _W))
    return out[0, 0].reshape(1)


# speedup vs baseline: 172.5473x; 1.1764x over previous
import jax
import jax.numpy as jnp
from jax.experimental import pallas as pl
from jax.experimental.pallas import tpu as pltpu

_H = 384
_W = 384
_N = float(_H * _W)
_R = _H // 8
_CPB = 4


def _top10_sum_fallback(x_ref, c, tau):
    def step(_, carry):
        bound, acc, rem = carry
        v = x_ref[c]
        w = jnp.where((v >= tau) & (v < bound), v, -jnp.inf)
        mx = jnp.max(w)
        cc = jnp.sum(jnp.where(w == mx, 1.0, 0.0))
        take = jnp.minimum(cc, rem)
        acc = acc + jnp.where(take > 0.0, take * mx, 0.0)
        rem = rem - take
        return mx, acc, rem

    _, acc, _ = jax.lax.fori_loop(
        0, 10, step,
        (jnp.float32(jnp.inf), jnp.float32(0.0), jnp.float32(10.0))
    )
    return acc


def _body(x_ref, gt_ref, m_ref, out_ref, m2_ref, m2g_ref, cef_ref):
    i = pl.program_id(0)

    @pl.when(i == 0)
    def _():
        m = m_ref[0]
        g = gt_ref[0]
        m2 = m * m
        m2g = m2 * g
        m2_ref[0] = m2
        m2g_ref[0] = m2g
        cef_ref[0] = jnp.sum(m2)
        cef_ref[1] = jnp.sum(m2g)
        cef_ref[2] = jnp.sum(m2g * g)
        out_ref[...] = jnp.zeros(out_ref.shape, jnp.float32)

    def chunk(j, carry):
        m2c = m2_ref[0, j]
        m2gc = m2g_ref[0, j]
        accs = []
        for c in range(_CPB):
            aS, aA, aB, aD, aM = carry[c]
            xv = x_ref[c, j]
            vm2 = xv * m2c
            aS = aS + xv
            aA = aA + xv * vm2
            aB = aB + vm2
            aD = aD + xv * m2gc
            aM = jnp.maximum(aM, xv)
            accs.append((aS, aA, aB, aD, aM))
        return tuple(accs)

    zero = jnp.zeros((8, _W), jnp.float32)
    init = tuple(
        (zero, zero, zero, zero,
         jnp.full((8, _W), -jnp.inf, jnp.float32))
        for _ in range(_CPB)
    )
    final = jax.lax.fori_loop(0, _R, chunk, init, unroll=True)

    sums = []
    Ws = []
    for c in range(_CPB):
        aS, aA, aB, aD, M = final[c]
        sums.append((jnp.sum(aS), jnp.sum(aA), jnp.sum(aB), jnp.sum(aD)))
        Ws.append(jnp.maximum(jnp.maximum(M[:, :128], M[:, 128:256]),
                              M[:, 256:]))

    def tau_step(_, carry):
        out = []
        for c in range(_CPB):
            W, _tau = carry[c]
            mx = jnp.max(W, axis=1, keepdims=True)
            mx = jnp.max(mx, axis=0, keepdims=True)
            mxb = jax.lax.broadcast_in_dim(mx, (8, 128), (0, 1))
            W = jnp.where(W == mxb, -jnp.inf, W)
            out.append((W, mxb))
        return tuple(out)

    taus_c = jax.lax.fori_loop(
        0, 10, tau_step,
        tuple((Ws[c], Ws[c]) for c in range(_CPB)),
        unroll=True,
    )
    tau_wide = [
        jnp.concatenate([taus_c[c][1]] * (_W // 128), axis=1)
        for c in range(_CPB)
    ]

    def fchunk(j, carry):
        out = []
        for c in range(_CPB):
            aC, aV = carry[c]
            xv = x_ref[c, j]
            sel = xv >= tau_wide[c]
            aC = aC + jnp.where(sel, 1.0, 0.0)
            aV = aV + jnp.where(sel, xv, 0.0)
            out.append((aC, aV))
        return tuple(out)

    facc = jax.lax.fori_loop(
        0, _R, fchunk, tuple((zero, zero) for _ in range(_CPB)),
        unroll=True,
    )

    C = cef_ref[0]
    E = cef_ref[1]
    F = cef_ref[2]

    cnts = [jnp.sum(facc[c][0]) for c in range(_CPB)]
    ssums = [jnp.sum(facc[c][1]) for c in range(_CPB)]

    all_exact = (cnts[0] == 10.0)
    for c in range(1, _CPB):
        all_exact = all_exact & (cnts[c] == 10.0)

    def _common(_):
        return tuple(ssums)

    def _rare(_):
        out = []
        for c in range(_CPB):
            tau_s = taus_c[c][1][0, 0]
            out.append(jax.lax.cond(
                cnts[c] == 10.0, lambda _, cc=c: ssums[cc],
                lambda _, cc=c, ts=tau_s: _top10_sum_fallback(x_ref, cc, ts),
                operand=None))
        return tuple(out)

    top10_sums = jax.lax.cond(all_exact, _common, _rare, operand=None)

    loss = jnp.float32(0.0)
    for c in range(_CPB):
        top10_sum = top10_sums[c]
        S, A, B, D = sums[c]
        mu = S / _N
        max_avg = top10_sum / 10.0 - mu
        denom = jnp.where(max_avg < 1e-20, max_avg + 1e-19, max_avg)
        num = ((A - 2.0 * mu * B + mu * mu * C) / denom
               - 2.0 * (D - mu * E)) / denom + F
        loss = loss + num / _N

    out_ref[...] += jnp.full(out_ref.shape, loss, dtype=jnp.float32)


@jax.jit
def kernel(pattern, pattern_gt, mask):
    ch = pattern.shape[1]
    x = pattern.reshape(ch, _R, 8, _W)
    out = pl.pallas_call(
        _body,
        grid=(ch // _CPB,),
        in_specs=[
            pl.BlockSpec((_CPB, _R, 8, _W), lambda i: (i, 0, 0, 0)),
            pl.BlockSpec((1, _R, 8, _W), lambda i: (0, 0, 0, 0)),
            pl.BlockSpec((1, _R, 8, _W), lambda i: (0, 0, 0, 0)),
        ],
        out_specs=pl.BlockSpec((8, 128), lambda i: (0, 0)),
        out_shape=jax.ShapeDtypeStruct((8, 128), jnp.float32),
        scratch_shapes=[
            pltpu.VMEM((1, _R, 8, _W), jnp.float32),
            pltpu.VMEM((1, _R, 8, _W), jnp.float32),
            pltpu.SMEM((3,), jnp.float32),
        ],
    )(x, pattern_gt.reshape(1, _R, 8, _W), mask.reshape(1, _R, 8, _W))
    return out[0, 0].reshape(1)


# speedup vs baseline: 191.7259x; 1.1111x over previous
import jax
import jax.numpy as jnp
from jax.experimental import pallas as pl
from jax.experimental.pallas import tpu as pltpu

_H = 384
_W = 384
_N = float(_H * _W)
_R = _H // 8
_CPB = 8


def _top10_sum_fallback(x_ref, c, tau):
    def step(_, carry):
        bound, acc, rem = carry
        v = x_ref[c]
        w = jnp.where((v >= tau) & (v < bound), v, -jnp.inf)
        mx = jnp.max(w)
        cc = jnp.sum(jnp.where(w == mx, 1.0, 0.0))
        take = jnp.minimum(cc, rem)
        acc = acc + jnp.where(take > 0.0, take * mx, 0.0)
        rem = rem - take
        return mx, acc, rem

    _, acc, _ = jax.lax.fori_loop(
        0, 10, step,
        (jnp.float32(jnp.inf), jnp.float32(0.0), jnp.float32(10.0))
    )
    return acc


def _body(x_ref, gt_ref, m_ref, out_ref, m2_ref, m2g_ref, cef_ref):
    i = pl.program_id(0)

    @pl.when(i == 0)
    def _():
        m = m_ref[0]
        g = gt_ref[0]
        m2 = m * m
        m2g = m2 * g
        m2_ref[0] = m2
        m2g_ref[0] = m2g
        cef_ref[0] = jnp.sum(m2)
        cef_ref[1] = jnp.sum(m2g)
        cef_ref[2] = jnp.sum(m2g * g)
        out_ref[...] = jnp.zeros(out_ref.shape, jnp.float32)

    def chunk(j, carry):
        m2c = m2_ref[0, j]
        m2gc = m2g_ref[0, j]
        accs = []
        for c in range(_CPB):
            aS, aA, aB, aD, aM = carry[c]
            xv = x_ref[c, j]
            vm2 = xv * m2c
            aS = aS + xv
            aA = aA + xv * vm2
            aB = aB + vm2
            aD = aD + xv * m2gc
            aM = jnp.maximum(aM, xv)
            accs.append((aS, aA, aB, aD, aM))
        return tuple(accs)

    zero = jnp.zeros((8, _W), jnp.float32)
    init = tuple(
        (zero, zero, zero, zero,
         jnp.full((8, _W), -jnp.inf, jnp.float32))
        for _ in range(_CPB)
    )
    final = jax.lax.fori_loop(0, _R, chunk, init, unroll=True)

    sums = []
    Ws = []
    for c in range(_CPB):
        aS, aA, aB, aD, M = final[c]
        sums.append((jnp.sum(aS), jnp.sum(aA), jnp.sum(aB), jnp.sum(aD)))
        Ws.append(jnp.maximum(jnp.maximum(M[:, :128], M[:, 128:256]),
                              M[:, 256:]))

    def tau_step(_, carry):
        out = []
        for c in range(_CPB):
            W, _tau = carry[c]
            mx = jnp.max(W, axis=1, keepdims=True)
            mx = jnp.max(mx, axis=0, keepdims=True)
            mxb = jax.lax.broadcast_in_dim(mx, (8, 128), (0, 1))
            W = jnp.where(W == mxb, -jnp.inf, W)
            out.append((W, mxb))
        return tuple(out)

    taus_c = jax.lax.fori_loop(
        0, 10, tau_step,
        tuple((Ws[c], Ws[c]) for c in range(_CPB)),
        unroll=True,
    )
    tau_wide = [
        jnp.concatenate([taus_c[c][1]] * (_W // 128), axis=1)
        for c in range(_CPB)
    ]

    def fchunk(j, carry):
        out = []
        for c in range(_CPB):
            aC, aV = carry[c]
            xv = x_ref[c, j]
            sel = xv >= tau_wide[c]
            aC = aC + jnp.where(sel, 1.0, 0.0)
            aV = aV + jnp.where(sel, xv, 0.0)
            out.append((aC, aV))
        return tuple(out)

    facc = jax.lax.fori_loop(
        0, _R, fchunk, tuple((zero, zero) for _ in range(_CPB)),
        unroll=True,
    )

    C = cef_ref[0]
    E = cef_ref[1]
    F = cef_ref[2]

    cnts = [jnp.sum(facc[c][0]) for c in range(_CPB)]
    ssums = [jnp.sum(facc[c][1]) for c in range(_CPB)]

    all_exact = (cnts[0] == 10.0)
    for c in range(1, _CPB):
        all_exact = all_exact & (cnts[c] == 10.0)

    def _common(_):
        return tuple(ssums)

    def _rare(_):
        out = []
        for c in range(_CPB):
            tau_s = taus_c[c][1][0, 0]
            out.append(jax.lax.cond(
                cnts[c] == 10.0, lambda _, cc=c: ssums[cc],
                lambda _, cc=c, ts=tau_s: _top10_sum_fallback(x_ref, cc, ts),
                operand=None))
        return tuple(out)

    top10_sums = jax.lax.cond(all_exact, _common, _rare, operand=None)

    loss = jnp.float32(0.0)
    for c in range(_CPB):
        top10_sum = top10_sums[c]
        S, A, B, D = sums[c]
        mu = S / _N
        max_avg = top10_sum / 10.0 - mu
        denom = jnp.where(max_avg < 1e-20, max_avg + 1e-19, max_avg)
        num = ((A - 2.0 * mu * B + mu * mu * C) / denom
               - 2.0 * (D - mu * E)) / denom + F
        loss = loss + num / _N

    out_ref[...] += jnp.full(out_ref.shape, loss, dtype=jnp.float32)


@jax.jit
def kernel(pattern, pattern_gt, mask):
    ch = pattern.shape[1]
    x = pattern.reshape(ch, _R, 8, _W)
    out = pl.pallas_call(
        _body,
        grid=(ch // _CPB,),
        in_specs=[
            pl.BlockSpec((_CPB, _R, 8, _W), lambda i: (i, 0, 0, 0)),
            pl.BlockSpec((1, _R, 8, _W), lambda i: (0, 0, 0, 0)),
            pl.BlockSpec((1, _R, 8, _W), lambda i: (0, 0, 0, 0)),
        ],
        out_specs=pl.BlockSpec((8, 128), lambda i: (0, 0)),
        out_shape=jax.ShapeDtypeStruct((8, 128), jnp.float32),
        scratch_shapes=[
            pltpu.VMEM((1, _R, 8, _W), jnp.float32),
            pltpu.VMEM((1, _R, 8, _W), jnp.float32),
            pltpu.SMEM((3,), jnp.float32),
        ],
    )(x, pattern_gt.reshape(1, _R, 8, _W), mask.reshape(1, _R, 8, _W))
    return out[0, 0].reshape(1)
